# baseline (speedup 1.0000x reference)
import dataclasses

import jax
import jax.numpy as jnp
from jax import lax
from jax.experimental import pallas as pl
from jax.experimental.pallas import tpu as pltpu
from jax.experimental.pallas import tpu_sc as plsc

_B = 4096
_D = 2048
_NREG = 64
_VR = 256

_BM = 512
_G = _B // _BM

_NC = 2
_NS = 16
_NW = _NC * _NS
_RPW = _B // _NW
_L = 16
_GRP = _RPW // _L


def _tc_body(x_ref, wr_ref, br_ref, ww_ref, bw_ref, wv_ref, bv_ref, emb_ref,
             vm_ref, ro_ref, wo_ref, vo_ref, ridx_ref, widx_ref, wval_ref,
             semb_ref):
    x = x_ref[...]

    def head(w_ref, b_ref):
        return jnp.dot(x, w_ref[...], preferred_element_type=jnp.float32) \
            + b_ref[...]

    def amax(l):
        m = jnp.max(l, axis=-1, keepdims=True)
        ii = lax.broadcasted_iota(jnp.int32, l.shape, 1)
        return jnp.min(jnp.where(l == m, ii, l.shape[1]), axis=-1,
                       keepdims=True).astype(jnp.int32)

    rl = head(wr_ref, br_ref)
    wl = head(ww_ref, bw_ref)
    vl = head(wv_ref, bv_ref)
    ro_ref[...] = rl
    wo_ref[...] = wl
    vo_ref[...] = vl
    ridx_ref[...] = amax(rl)
    widx_ref[...] = amax(wl)
    wval_ref[...] = amax(vl)

    @pl.when(pl.program_id(0) == 0)
    def _():
        semb_ref[...] = emb_ref[...] * vm_ref[0, 0]


def _tc_call(x, w_r, b_r, w_w, b_w, w_v, b_v, emb, vm):
    f32 = jnp.float32
    i32 = jnp.int32
    in_specs = [
        pl.BlockSpec((_BM, _D), lambda i: (i, 0)),
        pl.BlockSpec((_D, _NREG + 1), lambda i: (0, 0)),
        pl.BlockSpec((1, _NREG + 1), lambda i: (0, 0)),
        pl.BlockSpec((_D, _NREG + 1), lambda i: (0, 0)),
        pl.BlockSpec((1, _NREG + 1), lambda i: (0, 0)),
        pl.BlockSpec((_D, _VR), lambda i: (0, 0)),
        pl.BlockSpec((1, _VR), lambda i: (0, 0)),
        pl.BlockSpec((_VR, _D), lambda i: (0, 0)),
        pl.BlockSpec((1, 1), lambda i: (0, 0)),
    ]
    out_specs = [
        pl.BlockSpec((_BM, _NREG + 1), lambda i: (i, 0)),
        pl.BlockSpec((_BM, _NREG + 1), lambda i: (i, 0)),
        pl.BlockSpec((_BM, _VR), lambda i: (i, 0)),
        pl.BlockSpec((_BM, 1), lambda i: (i, 0)),
        pl.BlockSpec((_BM, 1), lambda i: (i, 0)),
        pl.BlockSpec((_BM, 1), lambda i: (i, 0)),
        pl.BlockSpec((_VR, _D), lambda i: (0, 0)),
    ]
    out_shape = [
        jax.ShapeDtypeStruct((_B, _NREG + 1), f32),
        jax.ShapeDtypeStruct((_B, _NREG + 1), f32),
        jax.ShapeDtypeStruct((_B, _VR), f32),
        jax.ShapeDtypeStruct((_B, 1), i32),
        jax.ShapeDtypeStruct((_B, 1), i32),
        jax.ShapeDtypeStruct((_B, 1), i32),
        jax.ShapeDtypeStruct((_VR, _D), f32),
    ]
    return pl.pallas_call(
        _tc_body,
        grid=(_G,),
        in_specs=in_specs,
        out_specs=out_specs,
        out_shape=out_shape,
        compiler_params=pltpu.CompilerParams(
            dimension_semantics=("arbitrary",)),
    )(x, w_r, b_r, w_w, b_w, w_v, b_v, emb, vm)


_NBUF = 3


def _sc_body(regs_hbm, ridx_hbm, widx_hbm, wval_hbm, semb_hbm,
             nregs_hbm, rv_hbm, fb_hbm,
             ridx_v, widx_v, wval_v, regs_v, rv_v, rvc_v,
             rows0_v, rows1_v, rows2_v,
             sem_idx, sem_regs,
             sin0, sin1, sin2, sout0, sout1, sout2):
    wid = lax.axis_index("s") * _NC + lax.axis_index("c")
    base = wid * _RPW
    c_ri = pltpu.async_copy(ridx_hbm.at[pl.ds(base, _RPW)], ridx_v, sem_idx)
    c_wi = pltpu.async_copy(widx_hbm.at[pl.ds(base, _RPW)], widx_v, sem_idx)
    c_wv = pltpu.async_copy(wval_hbm.at[pl.ds(base, _RPW)], wval_v, sem_idx)
    c_rg = pltpu.async_copy(regs_hbm.at[pl.ds(base, _RPW)], regs_v, sem_regs)
    c_ri.wait()
    c_wi.wait()
    c_wv.wait()
    c_rg.wait()

    rows = [rows0_v, rows1_v, rows2_v]
    sin = [sin0, sin1, sin2]
    sout = [sout0, sout1, sout2]
    in_h = [None] * _GRP
    out_h = [None] * _GRP

    def start_out(g):
        in_h[g].wait()
        out_h[g] = pltpu.async_copy(
            rows[g % _NBUF], fb_hbm.at[pl.ds(base + g * _L, _L)],
            sout[g % _NBUF])

    for g in range(_GRP):
        sl = pl.ds(g * _L, _L)
        ri = ridx_v[sl]
        wi = widx_v[sl]
        wv = wval_v[sl]
        rows16 = lax.iota(jnp.int32, _L) + (g * _L)
        rcol = jnp.minimum(ri, _NREG - 1)
        rval = plsc.load_gather(regs_v, [rows16, rcol])
        rval = jnp.where(ri == _NREG, 0, rval)
        rv_v[g, :] = rval
        rvc_v[g, :] = jnp.minimum(jnp.maximum(rval, 0), _VR - 1)
        wmask = wi < _NREG
        wcol = jnp.minimum(wi, _NREG - 1)
        plsc.store_scatter(regs_v, [rows16, wcol], wv, mask=wmask)

        if g >= _NBUF:
            out_h[g - _NBUF].wait()
        in_h[g] = pltpu.async_copy(
            semb_hbm.at[rvc_v.at[g]], rows[g % _NBUF], sin[g % _NBUF])
        if g >= 1:
            start_out(g - 1)

    c_nr = pltpu.async_copy(regs_v, nregs_hbm.at[pl.ds(base, _RPW)], sem_regs)
    c_rv = pltpu.async_copy(rv_v, rv_hbm.at[wid], sem_idx)

    start_out(_GRP - 1)
    for g in range(_GRP - _NBUF, _GRP):
        out_h[g].wait()
    c_nr.wait()
    c_rv.wait()


def _sc_call(registers, ridx, widx, wval, semb):
    i32 = jnp.int32
    f32 = jnp.float32
    mesh = plsc.VectorSubcoreMesh(core_axis_name="c", subcore_axis_name="s")
    cp = pltpu.CompilerParams()
    if "needs_layout_passes" in pltpu.CompilerParams.__dataclass_fields__:
        cp = dataclasses.replace(cp, needs_layout_passes=False)
    kern = pl.kernel(
        _sc_body,
        out_type=[
            jax.ShapeDtypeStruct((_B, _NREG), i32),
            jax.ShapeDtypeStruct((_NW, _GRP, _L), i32),
            jax.ShapeDtypeStruct((_B, _D), f32),
        ],
        mesh=mesh,
        scratch_types=[
            pltpu.VMEM((_RPW,), i32),
            pltpu.VMEM((_RPW,), i32),
            pltpu.VMEM((_RPW,), i32),
            pltpu.VMEM((_RPW, _NREG), i32),
            pltpu.VMEM((_GRP, _L), i32),
            pltpu.VMEM((_GRP, _L), i32),
            pltpu.VMEM((_L, _D), f32),
            pltpu.VMEM((_L, _D), f32),
            pltpu.VMEM((_L, _D), f32),
            pltpu.SemaphoreType.DMA,
            pltpu.SemaphoreType.DMA,
            pltpu.SemaphoreType.DMA,
            pltpu.SemaphoreType.DMA,
            pltpu.SemaphoreType.DMA,
            pltpu.SemaphoreType.DMA,
            pltpu.SemaphoreType.DMA,
            pltpu.SemaphoreType.DMA,
        ],
        compiler_params=cp,
    )
    return kern(registers, ridx, widx, wval, semb)


def kernel(x, registers, W_read, b_read, W_write, b_write, W_val, b_val,
           value_emb, value_mix):
    br = b_read.reshape(1, _NREG + 1)
    bw = b_write.reshape(1, _NREG + 1)
    bv = b_val.reshape(1, _VR)
    vm = value_mix.reshape(1, 1)
    ro, wo, vo, ridx, widx, wval, semb = _tc_call(
        x, W_read, br, W_write, bw, W_val, bv, value_emb, vm)
    nregs, rv, fb = _sc_call(
        registers, ridx.reshape(_B), widx.reshape(_B), wval.reshape(_B), semb)
    return (ro, wo, vo, nregs, rv.reshape(_B), fb)


# speedup vs baseline: 3.9306x; 3.9306x over previous
import dataclasses

import jax
import jax.numpy as jnp
from jax import lax
from jax.experimental import pallas as pl
from jax.experimental.pallas import tpu as pltpu
from jax.experimental.pallas import tpu_sc as plsc

_B = 4096
_D = 2048
_NREG = 64
_VR = 256

_BM = 512
_G = _B // _BM

_NC = 2
_NS = 16
_NW = _NC * _NS
_RPW = _B // _NW
_L = 16
_GRP = _RPW // _L


def _tc_body(x_ref, regs_ref, wr_ref, br_ref, ww_ref, bw_ref, wv_ref, bv_ref,
             emb_ref, vm_ref,
             ro_ref, wo_ref, vo_ref, widx_ref, wval_ref, rv_ref, fb_ref):
    x = x_ref[...]

    def head(w_ref, b_ref):
        return jnp.dot(x, w_ref[...], preferred_element_type=jnp.float32) \
            + b_ref[...]

    def amax(l):
        m = jnp.max(l, axis=-1, keepdims=True)
        ii = lax.broadcasted_iota(jnp.int32, l.shape, 1)
        return jnp.min(jnp.where(l == m, ii, l.shape[1]), axis=-1,
                       keepdims=True).astype(jnp.int32)

    rl = head(wr_ref, br_ref)
    wl = head(ww_ref, bw_ref)
    vl = head(wv_ref, bv_ref)
    ro_ref[...] = rl
    wo_ref[...] = wl
    vo_ref[...] = vl
    ridx = amax(rl)
    widx_ref[...] = amax(wl)
    wval_ref[...] = amax(vl)

    regs = regs_ref[...]
    col = lax.broadcasted_iota(jnp.int32, regs.shape, 1)
    rv = jnp.sum(jnp.where(col == ridx, regs, 0), axis=-1, keepdims=True)
    rv_ref[...] = rv

    rvc = jnp.minimum(jnp.maximum(rv, 0), _VR - 1)
    vcol = lax.broadcasted_iota(jnp.int32, (rv.shape[0], _VR), 1)
    onehot = (vcol == rvc).astype(jnp.float32)
    fb_ref[...] = vm_ref[0, 0] * jnp.dot(
        onehot, emb_ref[...], preferred_element_type=jnp.float32)


def _tc_call(x, registers, w_r, b_r, w_w, b_w, w_v, b_v, emb, vm):
    f32 = jnp.float32
    i32 = jnp.int32
    in_specs = [
        pl.BlockSpec((_BM, _D), lambda i: (i, 0)),
        pl.BlockSpec((_BM, _NREG), lambda i: (i, 0)),
        pl.BlockSpec((_D, _NREG + 1), lambda i: (0, 0)),
        pl.BlockSpec((1, _NREG + 1), lambda i: (0, 0)),
        pl.BlockSpec((_D, _NREG + 1), lambda i: (0, 0)),
        pl.BlockSpec((1, _NREG + 1), lambda i: (0, 0)),
        pl.BlockSpec((_D, _VR), lambda i: (0, 0)),
        pl.BlockSpec((1, _VR), lambda i: (0, 0)),
        pl.BlockSpec((_VR, _D), lambda i: (0, 0)),
        pl.BlockSpec((1, 1), lambda i: (0, 0)),
    ]
    out_specs = [
        pl.BlockSpec((_BM, _NREG + 1), lambda i: (i, 0)),
        pl.BlockSpec((_BM, _NREG + 1), lambda i: (i, 0)),
        pl.BlockSpec((_BM, _VR), lambda i: (i, 0)),
        pl.BlockSpec((_BM, 1), lambda i: (i, 0)),
        pl.BlockSpec((_BM, 1), lambda i: (i, 0)),
        pl.BlockSpec((_BM, 1), lambda i: (i, 0)),
        pl.BlockSpec((_BM, _D), lambda i: (i, 0)),
    ]
    out_shape = [
        jax.ShapeDtypeStruct((_B, _NREG + 1), f32),
        jax.ShapeDtypeStruct((_B, _NREG + 1), f32),
        jax.ShapeDtypeStruct((_B, _VR), f32),
        jax.ShapeDtypeStruct((_B, 1), i32),
        jax.ShapeDtypeStruct((_B, 1), i32),
        jax.ShapeDtypeStruct((_B, 1), i32),
        jax.ShapeDtypeStruct((_B, _D), f32),
    ]
    return pl.pallas_call(
        _tc_body,
        grid=(_G,),
        in_specs=in_specs,
        out_specs=out_specs,
        out_shape=out_shape,
        compiler_params=pltpu.CompilerParams(
            dimension_semantics=("arbitrary",)),
    )(x, registers, w_r, b_r, w_w, b_w, w_v, b_v, emb, vm)


def _sc_body(regs_hbm, widx_hbm, wval_hbm, nregs_hbm,
             widx_v, wval_v, regs_v, sem_idx, sem_regs):
    wid = lax.axis_index("s") * _NC + lax.axis_index("c")
    base = wid * _RPW
    c_wi = pltpu.async_copy(widx_hbm.at[pl.ds(base, _RPW)], widx_v, sem_idx)
    c_wv = pltpu.async_copy(wval_hbm.at[pl.ds(base, _RPW)], wval_v, sem_idx)
    c_rg = pltpu.async_copy(regs_hbm.at[pl.ds(base, _RPW)], regs_v, sem_regs)
    c_wi.wait()
    c_wv.wait()
    c_rg.wait()

    for g in range(_GRP):
        sl = pl.ds(g * _L, _L)
        wi = widx_v[sl]
        wv = wval_v[sl]
        rows16 = lax.iota(jnp.int32, _L) + (g * _L)
        wmask = wi < _NREG
        wcol = jnp.minimum(wi, _NREG - 1)
        plsc.store_scatter(regs_v, [rows16, wcol], wv, mask=wmask)

    pltpu.sync_copy(regs_v, nregs_hbm.at[pl.ds(base, _RPW)])


def _sc_call(registers, widx, wval):
    i32 = jnp.int32
    mesh = plsc.VectorSubcoreMesh(core_axis_name="c", subcore_axis_name="s")
    cp = pltpu.CompilerParams()
    if "needs_layout_passes" in pltpu.CompilerParams.__dataclass_fields__:
        cp = dataclasses.replace(cp, needs_layout_passes=False)
    kern = pl.kernel(
        _sc_body,
        out_type=jax.ShapeDtypeStruct((_B, _NREG), i32),
        mesh=mesh,
        scratch_types=[
            pltpu.VMEM((_RPW,), i32),
            pltpu.VMEM((_RPW,), i32),
            pltpu.VMEM((_RPW, _NREG), i32),
            pltpu.SemaphoreType.DMA,
            pltpu.SemaphoreType.DMA,
        ],
        compiler_params=cp,
    )
    return kern(registers, widx, wval)


def kernel(x, registers, W_read, b_read, W_write, b_write, W_val, b_val,
           value_emb, value_mix):
    br = b_read.reshape(1, _NREG + 1)
    bw = b_write.reshape(1, _NREG + 1)
    bv = b_val.reshape(1, _VR)
    vm = value_mix.reshape(1, 1)
    ro, wo, vo, widx, wval, rv, fb = _tc_call(
        x, registers, W_read, br, W_write, bw, W_val, bv, value_emb, vm)
    nregs = _sc_call(registers, widx.reshape(_B), wval.reshape(_B))
    return (ro, wo, vo, nregs, rv.reshape(_B), fb)
